# Initial kernel scaffold; baseline (speedup 1.0000x reference)
import jax
import jax.numpy as jnp
from jax.experimental import pallas as pl


def kernel(x, edge_index, W1, b1, gamma, beta, W2, b2):
    raise NotImplementedError("write your pallas kernel here")


# speedup vs baseline: 3.4353x; 3.4353x over previous
import functools

import jax
import jax.numpy as jnp
from jax import lax
from jax.experimental import pallas as pl
from jax.experimental.pallas import tpu as pltpu
from jax.experimental.pallas import tpu_sc as plsc

N = 10000
E = 320000
F = 128
H = 128
BN_EPS = 1e-5

NC = 2
NS = 16
NW = NC * NS
LANES = 128
EPT_ROWS = 80
EPT = EPT_ROWS * LANES
E_PAD = NW * EPT
N_SH = 10240
ZROWS = N_SH // NS


def _make_sc_aggregate():
    mesh = plsc.VectorSubcoreMesh(core_axis_name="c", subcore_axis_name="s")

    @functools.partial(
        pl.kernel,
        mesh=mesh,
        out_type=jax.ShapeDtypeStruct((NC, N_SH, F), jnp.float32),
        scratch_types=[
            pltpu.VMEM((EPT_ROWS, LANES), jnp.int32),
            pltpu.VMEM((EPT_ROWS, LANES), jnp.int32),
            pltpu.VMEM((LANES, F), jnp.float32),
            pltpu.VMEM_SHARED((N_SH, F), jnp.float32),
            pltpu.SemaphoreType.DMA,
        ],
    )
    def agg(x_hbm, src_hbm, dst_hbm, z_hbm, out_hbm,
            src_v, dst_v, rows_v, aggr_sh, sem):
        c = lax.axis_index("c")
        s = lax.axis_index("s")
        wid = s * NC + c

        pltpu.sync_copy(z_hbm.at[pl.ds(s * ZROWS, ZROWS)],
                        aggr_sh.at[pl.ds(s * ZROWS, ZROWS)])
        pltpu.sync_copy(src_hbm.at[wid], src_v)
        pltpu.sync_copy(dst_hbm.at[wid], dst_v)
        plsc.subcore_barrier()

        def body(j, carry):
            pltpu.async_copy(x_hbm.at[src_v.at[j]], rows_v, sem).wait()
            pltpu.sync_copy(rows_v, aggr_sh.at[dst_v.at[j]], add=True)
            return carry

        lax.fori_loop(0, EPT_ROWS, body, 0, unroll=False)
        plsc.subcore_barrier()
        pltpu.sync_copy(aggr_sh.at[pl.ds(s * ZROWS, ZROWS)],
                        out_hbm.at[c, pl.ds(s * ZROWS, ZROWS)])

    return agg


_sc_aggregate = _make_sc_aggregate()


def _mlp_body(x_ref, p_ref, w1_ref, b1_ref, g_ref, be_ref, w2_ref, b2_ref,
              o_ref):
    h = x_ref[...] + p_ref[0, :N, :] + p_ref[1, :N, :]
    h = jnp.dot(h, w1_ref[...], preferred_element_type=jnp.float32)
    h = jnp.maximum(h + b1_ref[...], 0.0)
    mean = jnp.mean(h, axis=0, keepdims=True)
    cent = h - mean
    var = jnp.mean(cent * cent, axis=0, keepdims=True)
    hn = cent * (jax.lax.rsqrt(var + BN_EPS) * g_ref[...]) + be_ref[...]
    o_ref[...] = (jnp.dot(hn, w2_ref[...], preferred_element_type=jnp.float32)
                  + b2_ref[...])


def _tc_mlp(x, partials, W1, b1, gamma, beta, W2, b2):
    return pl.pallas_call(
        _mlp_body,
        out_shape=jax.ShapeDtypeStruct((N, F), jnp.float32),
    )(x, partials, W1, b1.reshape(1, H), gamma.reshape(1, H),
      beta.reshape(1, H), W2, b2.reshape(1, H))


def kernel(x, edge_index, W1, b1, gamma, beta, W2, b2):
    src = edge_index[0]
    dst = edge_index[1]
    pad = E_PAD - E
    src_p = jnp.concatenate([src, jnp.zeros((pad,), jnp.int32)])
    dst_p = jnp.concatenate([dst, jnp.full((pad,), N, jnp.int32)])
    src3 = src_p.reshape(NW, EPT_ROWS, LANES)
    dst3 = dst_p.reshape(NW, EPT_ROWS, LANES)
    zeros = jnp.zeros((N_SH, F), jnp.float32)
    partials = _sc_aggregate(x, src3, dst3, zeros)
    return _tc_mlp(x, partials, W1, b1, gamma, beta, W2, b2)
